# Initial kernel scaffold; baseline (speedup 1.0000x reference)
import jax
import jax.numpy as jnp
from jax.experimental import pallas as pl


def kernel(x, table, W1, b1, W2, b2, W3, b3):
    raise NotImplementedError("write your pallas kernel here")


# speedup vs baseline: 2.4343x; 2.4343x over previous
import functools

import jax
import jax.numpy as jnp
from jax import lax
from jax.experimental import pallas as pl
from jax.experimental.pallas import tpu as pltpu
from jax.experimental.pallas import tpu_sc as plsc

B = 16384
L = 200
EMB = 64
H1 = 512
H2 = 256
OUT = 128

NC = 2
NS = 16
NW = NC * NS
SEQ_PER_W = B // NW
SBLK = 8
NBLK = SEQ_PER_W // SBLK
HALF = L // 2
LANES = 16
VPR = EMB // LANES


def _accum(rows_ref, acc):
    U = 5

    def body(i, acc):
        base = i * U
        cols = []
        for c in range(VPR):
            t = rows_ref[base, pl.ds(c * LANES, LANES)]
            for k in range(1, U):
                t = t + rows_ref[base + k, pl.ds(c * LANES, LANES)]
            cols.append(acc[c] + t)
        return tuple(cols)

    return lax.fori_loop(0, HALF // U, body, acc)


def _make_pool():
    mesh = plsc.VectorSubcoreMesh(
        core_axis_name="c", subcore_axis_name="s",
        num_cores=NC, num_subcores=NS)

    @functools.partial(
        pl.kernel,
        out_type=jax.ShapeDtypeStruct((B * EMB,), jnp.float32),
        mesh=mesh,
        scratch_types=[
            pltpu.VMEM((2 * SBLK, HALF), jnp.int32),
            pltpu.VMEM((HALF, EMB), jnp.float32),
            pltpu.VMEM((HALF, EMB), jnp.float32),
            pltpu.VMEM((SBLK * EMB,), jnp.float32),
            pltpu.SemaphoreType.DMA,
            pltpu.SemaphoreType.DMA,
        ],
        compiler_params=pltpu.CompilerParams(use_tc_tiling_on_sc=False),
    )
    def pool(x_hbm, table_hbm, out_hbm, idx_v, rows0, rows1, out_v, sem0, sem1):
        wid = lax.axis_index("s") * NC + lax.axis_index("c")
        seq_base = wid * SEQ_PER_W
        rows = (rows0, rows1)
        sems = (sem0, sem1)

        def block(blk, carry):
            seq0 = seq_base + blk * SBLK
            pltpu.sync_copy(x_hbm.at[pl.ds(seq0 * 2, 2 * SBLK)], idx_v)
            copies = [None] * (2 * SBLK)
            copies[0] = pltpu.async_copy(
                table_hbm.at[idx_v.at[0]], rows0, sem0)
            acc = None
            for h in range(2 * SBLK):
                if h + 1 < 2 * SBLK:
                    copies[h + 1] = pltpu.async_copy(
                        table_hbm.at[idx_v.at[h + 1]],
                        rows[(h + 1) % 2], sems[(h + 1) % 2])
                copies[h].wait()
                if h % 2 == 0:
                    acc = tuple(jnp.zeros((LANES,), jnp.float32)
                                for _ in range(VPR))
                acc = _accum(rows[h % 2], acc)
                if h % 2 == 1:
                    s = h // 2
                    for c in range(VPR):
                        out_v[pl.ds(s * EMB + c * LANES, LANES)] = (
                            acc[c] * (1.0 / L))
            pltpu.sync_copy(out_v,
                            out_hbm.at[pl.ds(seq0 * EMB, SBLK * EMB)])
            return carry

        lax.fori_loop(0, NBLK, block, 0)

    return pool


_pool = _make_pool()


def _mlp(pooled, W1, b1, W2, b2, W3, b3):
    BM = 2048

    def body(x_ref, w1, b1r, w2, b2r, w3, b3r, o_ref):
        h = jnp.dot(x_ref[...], w1[...],
                    preferred_element_type=jnp.float32) + b1r[...]
        h = jnp.maximum(h, 0.0)
        h = jnp.dot(h, w2[...], preferred_element_type=jnp.float32) + b2r[...]
        h = jnp.maximum(h, 0.0)
        o_ref[...] = jnp.dot(h, w3[...],
                             preferred_element_type=jnp.float32) + b3r[...]

    return pl.pallas_call(
        body,
        grid=(B // BM,),
        in_specs=[
            pl.BlockSpec((BM, EMB), lambda i: (i, 0)),
            pl.BlockSpec((EMB, H1), lambda i: (0, 0)),
            pl.BlockSpec((1, H1), lambda i: (0, 0)),
            pl.BlockSpec((H1, H2), lambda i: (0, 0)),
            pl.BlockSpec((1, H2), lambda i: (0, 0)),
            pl.BlockSpec((H2, OUT), lambda i: (0, 0)),
            pl.BlockSpec((1, OUT), lambda i: (0, 0)),
        ],
        out_specs=pl.BlockSpec((BM, OUT), lambda i: (i, 0)),
        out_shape=jax.ShapeDtypeStruct((B, OUT), jnp.float32),
    )(pooled, W1, b1, W2, b2, W3, b3)


def kernel(x, table, W1, b1, W2, b2, W3, b3):
    x2 = x.reshape(2 * B, HALF)
    pooled = _pool(x2, table).reshape(B, EMB)
    return _mlp(pooled, W1, b1.reshape(1, H1), W2, b2.reshape(1, H2),
                W3, b3.reshape(1, OUT))
